# baseline (speedup 1.0000x reference)
import functools

import jax
import jax.numpy as jnp
from jax import lax
from jax.experimental import pallas as pl
from jax.experimental.pallas import tpu as pltpu
from jax.experimental.pallas import tpu_sc as plsc

_N = 10000
_E = 160000
_A = 8
_DIM = 128

_NUM_WORKERS = 32
_CHUNK = _E // _NUM_WORKERS
_CPAD = (_CHUNK + 15) // 16 * 16

_BE = 6400


def _sc_codes(joint_acts, edge_index):
    mesh = plsc.VectorSubcoreMesh(core_axis_name="c", subcore_axis_name="s")

    @functools.partial(
        pl.kernel,
        mesh=mesh,
        compiler_params=pltpu.CompilerParams(needs_layout_passes=False),
        out_type=[
            jax.ShapeDtypeStruct((_E,), jnp.int32),
            jax.ShapeDtypeStruct((_E,), jnp.int32),
        ],
        scratch_types=[
            pltpu.VMEM((_N,), jnp.int32),
            pltpu.VMEM((_CPAD,), jnp.int32),
            pltpu.VMEM((_CPAD,), jnp.int32),
            pltpu.VMEM((_CPAD,), jnp.int32),
            pltpu.VMEM((_CPAD,), jnp.int32),
        ],
    )
    def k(ja_hbm, ei_hbm, c_hbm, cr_hbm, ja_v, src_v, dst_v, c_v, cr_v):
        wid = lax.axis_index("s") * 2 + lax.axis_index("c")
        base = wid * _CHUNK
        src_v[pl.ds(_CPAD - 16, 16)] = jnp.zeros((16,), jnp.int32)
        dst_v[pl.ds(_CPAD - 16, 16)] = jnp.zeros((16,), jnp.int32)
        pltpu.sync_copy(ja_hbm, ja_v)
        pltpu.sync_copy(ei_hbm.at[pl.ds(base, _CHUNK)], src_v.at[pl.ds(0, _CHUNK)])
        pltpu.sync_copy(ei_hbm.at[pl.ds(_E + base, _CHUNK)], dst_v.at[pl.ds(0, _CHUNK)])

        def body(i, carry):
            sl = pl.ds(i * 16, 16)
            a_s = plsc.load_gather(ja_v, [src_v[sl]])
            a_d = plsc.load_gather(ja_v, [dst_v[sl]])
            c_v[sl] = a_s * _A + a_d
            cr_v[sl] = a_d * _A + a_s
            return carry

        lax.fori_loop(0, _CPAD // 16, body, 0, unroll=4)
        pltpu.sync_copy(c_v.at[pl.ds(0, _CHUNK)], c_hbm.at[pl.ds(base, _CHUNK)])
        pltpu.sync_copy(cr_v.at[pl.ds(0, _CHUNK)], cr_hbm.at[pl.ds(base, _CHUNK)])

    return k(joint_acts, edge_index.reshape(2 * _E))


def _edge_body(xu_ref, xr_ref, c_ref, cr_ref,
               w1_ref, b1_ref, w3_ref, b3_ref, w2_ref, b2_ref, out_ref):
    @pl.when(pl.program_id(0) == 0)
    def _init():
        out_ref[...] = jnp.zeros_like(out_ref)

    diag = (lax.broadcasted_iota(jnp.int32, (_A * _A, _A * _A), 0)
            == lax.broadcasted_iota(jnp.int32, (_A * _A, _A * _A), 1))

    def stream(x, idx_row):
        g = jnp.maximum(
            jnp.dot(x, w1_ref[...], preferred_element_type=jnp.float32)
            + b1_ref[...], 0.0)
        g = jnp.maximum(
            jnp.dot(g, w3_ref[...], preferred_element_type=jnp.float32)
            + b3_ref[...], 0.0)
        logit = (jnp.dot(g, w2_ref[...], preferred_element_type=jnp.float32)
                 + b2_ref[...])
        onehot_t = jnp.where(
            idx_row == lax.broadcasted_iota(jnp.int32, (_A * _A, _BE), 0),
            1.0, 0.0)
        prod = jnp.dot(onehot_t, logit, preferred_element_type=jnp.float32,
                       precision=lax.Precision.HIGHEST)
        return jnp.sum(jnp.where(diag, prod, 0.0))

    base = pl.multiple_of(pl.program_id(0) * _BE, 128)
    part = (stream(xu_ref[...], c_ref[pl.ds(base, _BE)].reshape(1, _BE))
            + stream(xr_ref[...], cr_ref[pl.ds(base, _BE)].reshape(1, _BE)))
    out_ref[...] = out_ref[...] + part


def _node_body(x_ref, ja_ref, edge_ref, w1_ref, b1_ref, w3_ref, b3_ref,
               w2_ref, b2_ref, out_ref):
    h = jnp.maximum(
        jnp.dot(x_ref[...], w1_ref[...], preferred_element_type=jnp.float32)
        + b1_ref[...], 0.0)
    h = jnp.maximum(
        jnp.dot(h, w3_ref[...], preferred_element_type=jnp.float32)
        + b3_ref[...], 0.0)
    logit = (jnp.dot(h, w2_ref[...], preferred_element_type=jnp.float32)
             + b2_ref[...])
    onehot_t = jnp.where(
        ja_ref[...].reshape(1, _N)
        == lax.broadcasted_iota(jnp.int32, (_A, _N), 0), 1.0, 0.0)
    prod = jnp.dot(onehot_t, logit, preferred_element_type=jnp.float32, precision=lax.Precision.HIGHEST)
    diag = (lax.broadcasted_iota(jnp.int32, (_A, _A), 0)
            == lax.broadcasted_iota(jnp.int32, (_A, _A), 1))
    out_ref[...] = (jnp.sum(jnp.where(diag, prod, 0.0)).reshape(1, 1)
                    + 0.25 * edge_ref[...])


def kernel(edge_index, joint_acts, edge_feats_u, node_feats_u,
           edge_feat_reflected_u, W_ju1, b_ju1, W_ju3, b_ju3, W_ju2, b_ju2,
           W_iu1, b_iu1, W_iu3, b_iu3, W_iu2, b_iu2):
    c_flat, cr_flat = _sc_codes(joint_acts, edge_index)

    wfull = lambda shape: pl.BlockSpec(shape, lambda i: (0,) * len(shape))
    edge_out = pl.pallas_call(
        _edge_body,
        grid=(_E // _BE,),
        in_specs=[
            pl.BlockSpec((_BE, 3 * _DIM), lambda i: (i, 0)),
            pl.BlockSpec((_BE, 3 * _DIM), lambda i: (i, 0)),
            pl.BlockSpec((_E,), lambda i: (0,)),
            pl.BlockSpec((_E,), lambda i: (0,)),
            wfull((3 * _DIM, 32)),
            wfull((1, 32)),
            wfull((32, 32)),
            wfull((1, 32)),
            wfull((32, _A * _A)),
            wfull((1, _A * _A)),
        ],
        out_specs=pl.BlockSpec((1, 1), lambda i: (0, 0)),
        out_shape=jax.ShapeDtypeStruct((1, 1), jnp.float32),
    )(edge_feats_u, edge_feat_reflected_u, c_flat, cr_flat,
      W_ju1, b_ju1.reshape(1, -1), W_ju3, b_ju3.reshape(1, -1),
      W_ju2, b_ju2.reshape(1, -1))

    node_out = pl.pallas_call(
        _node_body,
        grid=(1,),
        in_specs=[
            pl.BlockSpec((_N, 2 * _DIM), lambda i: (0, 0)),
            pl.BlockSpec((1, 1, _N), lambda i: (0, 0, 0)),
            wfull((1, 1)),
            wfull((2 * _DIM, 32)),
            wfull((1, 32)),
            wfull((32, 32)),
            wfull((1, 32)),
            wfull((32, _A)),
            wfull((1, _A)),
        ],
        out_specs=pl.BlockSpec((1, 1), lambda i: (0, 0)),
        out_shape=jax.ShapeDtypeStruct((1, 1), jnp.float32),
    )(node_feats_u, joint_acts.reshape(1, 1, _N), edge_out,
      W_iu1, b_iu1.reshape(1, -1), W_iu3, b_iu3.reshape(1, -1),
      W_iu2, b_iu2.reshape(1, -1))

    return node_out


# speedup vs baseline: 1.3132x; 1.3132x over previous
import functools

import jax
import jax.numpy as jnp
from jax import lax
from jax.experimental import pallas as pl
from jax.experimental.pallas import tpu as pltpu
from jax.experimental.pallas import tpu_sc as plsc

_N = 10000
_E = 160000
_A = 8
_DIM = 128

_NUM_WORKERS = 32
_CHUNK = _E // _NUM_WORKERS
_CPAD = (_CHUNK + 15) // 16 * 16

_BE = 6400


def _sc_codes(joint_acts, edge_index):
    mesh = plsc.VectorSubcoreMesh(core_axis_name="c", subcore_axis_name="s")

    @functools.partial(
        pl.kernel,
        mesh=mesh,
        compiler_params=pltpu.CompilerParams(needs_layout_passes=False),
        out_type=[
            jax.ShapeDtypeStruct((_E,), jnp.int32),
            jax.ShapeDtypeStruct((_E,), jnp.int32),
        ],
        scratch_types=[
            pltpu.VMEM((_N,), jnp.int32),
            pltpu.VMEM((_CPAD,), jnp.int32),
            pltpu.VMEM((_CPAD,), jnp.int32),
            pltpu.VMEM((_CPAD,), jnp.int32),
            pltpu.VMEM((_CPAD,), jnp.int32),
        ],
    )
    def k(ja_hbm, ei_hbm, c_hbm, cr_hbm, ja_v, src_v, dst_v, c_v, cr_v):
        wid = lax.axis_index("s") * 2 + lax.axis_index("c")
        base = wid * _CHUNK
        src_v[pl.ds(_CPAD - 16, 16)] = jnp.zeros((16,), jnp.int32)
        dst_v[pl.ds(_CPAD - 16, 16)] = jnp.zeros((16,), jnp.int32)
        pltpu.sync_copy(ja_hbm, ja_v)
        pltpu.sync_copy(ei_hbm.at[pl.ds(base, _CHUNK)], src_v.at[pl.ds(0, _CHUNK)])
        pltpu.sync_copy(ei_hbm.at[pl.ds(_E + base, _CHUNK)], dst_v.at[pl.ds(0, _CHUNK)])

        def body(i, carry):
            sl = pl.ds(i * 16, 16)
            a_s = plsc.load_gather(ja_v, [src_v[sl]])
            a_d = plsc.load_gather(ja_v, [dst_v[sl]])
            c_v[sl] = a_s * _A + a_d
            cr_v[sl] = a_d * _A + a_s
            return carry

        lax.fori_loop(0, _CPAD // 16, body, 0, unroll=4)
        pltpu.sync_copy(c_v.at[pl.ds(0, _CHUNK)], c_hbm.at[pl.ds(base, _CHUNK)])
        pltpu.sync_copy(cr_v.at[pl.ds(0, _CHUNK)], cr_hbm.at[pl.ds(base, _CHUNK)])

    return k(joint_acts, edge_index.reshape(2 * _E))


def _edge_body(xu_ref, xr_ref, c_ref, cr_ref,
               w1_ref, b1_ref, w3_ref, b3_ref, w2_ref, b2_ref, out_ref):
    @pl.when(pl.program_id(0) == 0)
    def _init():
        out_ref[...] = jnp.zeros_like(out_ref)

    diag = (lax.broadcasted_iota(jnp.int32, (_A * _A, _A * _A), 0)
            == lax.broadcasted_iota(jnp.int32, (_A * _A, _A * _A), 1))

    def stream(x, idx_row):
        g = jnp.maximum(
            jnp.dot(x, w1_ref[...], preferred_element_type=jnp.float32)
            + b1_ref[...], 0.0)
        g = jnp.maximum(
            jnp.dot(g, w3_ref[...], preferred_element_type=jnp.float32)
            + b3_ref[...], 0.0)
        logit = (jnp.dot(g, w2_ref[...], preferred_element_type=jnp.float32)
                 + b2_ref[...])
        onehot_t = jnp.where(
            idx_row == lax.broadcasted_iota(jnp.int32, (_A * _A, _BE), 0),
            1.0, 0.0)
        hi = logit.astype(jnp.bfloat16).astype(jnp.float32)
        lo = logit - hi
        prod = (jnp.dot(onehot_t, hi, preferred_element_type=jnp.float32)
                + jnp.dot(onehot_t, lo, preferred_element_type=jnp.float32))
        return jnp.sum(jnp.where(diag, prod, 0.0))

    base = pl.multiple_of(pl.program_id(0) * _BE, 128)
    part = (stream(xu_ref[...], c_ref[pl.ds(base, _BE)].reshape(1, _BE))
            + stream(xr_ref[...], cr_ref[pl.ds(base, _BE)].reshape(1, _BE)))
    out_ref[...] = out_ref[...] + part


def _node_body(x_ref, ja_ref, edge_ref, w1_ref, b1_ref, w3_ref, b3_ref,
               w2_ref, b2_ref, out_ref):
    h = jnp.maximum(
        jnp.dot(x_ref[...], w1_ref[...], preferred_element_type=jnp.float32)
        + b1_ref[...], 0.0)
    h = jnp.maximum(
        jnp.dot(h, w3_ref[...], preferred_element_type=jnp.float32)
        + b3_ref[...], 0.0)
    logit = (jnp.dot(h, w2_ref[...], preferred_element_type=jnp.float32)
             + b2_ref[...])
    onehot_t = jnp.where(
        ja_ref[...].reshape(1, _N)
        == lax.broadcasted_iota(jnp.int32, (_A, _N), 0), 1.0, 0.0)
    hi = logit.astype(jnp.bfloat16).astype(jnp.float32)
    lo = logit - hi
    prod = (jnp.dot(onehot_t, hi, preferred_element_type=jnp.float32)
            + jnp.dot(onehot_t, lo, preferred_element_type=jnp.float32))
    diag = (lax.broadcasted_iota(jnp.int32, (_A, _A), 0)
            == lax.broadcasted_iota(jnp.int32, (_A, _A), 1))
    out_ref[...] = (jnp.sum(jnp.where(diag, prod, 0.0)).reshape(1, 1)
                    + 0.25 * edge_ref[...])


def kernel(edge_index, joint_acts, edge_feats_u, node_feats_u,
           edge_feat_reflected_u, W_ju1, b_ju1, W_ju3, b_ju3, W_ju2, b_ju2,
           W_iu1, b_iu1, W_iu3, b_iu3, W_iu2, b_iu2):
    c_flat, cr_flat = _sc_codes(joint_acts, edge_index)

    wfull = lambda shape: pl.BlockSpec(shape, lambda i: (0,) * len(shape))
    edge_out = pl.pallas_call(
        _edge_body,
        grid=(_E // _BE,),
        in_specs=[
            pl.BlockSpec((_BE, 3 * _DIM), lambda i: (i, 0)),
            pl.BlockSpec((_BE, 3 * _DIM), lambda i: (i, 0)),
            pl.BlockSpec((_E,), lambda i: (0,)),
            pl.BlockSpec((_E,), lambda i: (0,)),
            wfull((3 * _DIM, 32)),
            wfull((1, 32)),
            wfull((32, 32)),
            wfull((1, 32)),
            wfull((32, _A * _A)),
            wfull((1, _A * _A)),
        ],
        out_specs=pl.BlockSpec((1, 1), lambda i: (0, 0)),
        out_shape=jax.ShapeDtypeStruct((1, 1), jnp.float32),
    )(edge_feats_u, edge_feat_reflected_u, c_flat, cr_flat,
      W_ju1, b_ju1.reshape(1, -1), W_ju3, b_ju3.reshape(1, -1),
      W_ju2, b_ju2.reshape(1, -1))

    node_out = pl.pallas_call(
        _node_body,
        grid=(1,),
        in_specs=[
            pl.BlockSpec((_N, 2 * _DIM), lambda i: (0, 0)),
            pl.BlockSpec((1, 1, _N), lambda i: (0, 0, 0)),
            wfull((1, 1)),
            wfull((2 * _DIM, 32)),
            wfull((1, 32)),
            wfull((32, 32)),
            wfull((1, 32)),
            wfull((32, _A)),
            wfull((1, _A)),
        ],
        out_specs=pl.BlockSpec((1, 1), lambda i: (0, 0)),
        out_shape=jax.ShapeDtypeStruct((1, 1), jnp.float32),
    )(node_feats_u, joint_acts.reshape(1, 1, _N), edge_out,
      W_iu1, b_iu1.reshape(1, -1), W_iu3, b_iu3.reshape(1, -1),
      W_iu2, b_iu2.reshape(1, -1))

    return node_out


# speedup vs baseline: 1.4393x; 1.0960x over previous
import functools

import jax
import jax.numpy as jnp
from jax import lax
from jax.experimental import pallas as pl
from jax.experimental.pallas import tpu as pltpu
from jax.experimental.pallas import tpu_sc as plsc

_N = 10000
_E = 160000
_A = 8
_DIM = 128

_NUM_WORKERS = 32
_CHUNK = _E // _NUM_WORKERS
_CPAD = (_CHUNK + 15) // 16 * 16

_BE = 6400


def _sc_codes(joint_acts, edge_index):
    mesh = plsc.VectorSubcoreMesh(core_axis_name="c", subcore_axis_name="s")

    @functools.partial(
        pl.kernel,
        mesh=mesh,
        compiler_params=pltpu.CompilerParams(needs_layout_passes=False),
        out_type=[
            jax.ShapeDtypeStruct((_E,), jnp.int32),
            jax.ShapeDtypeStruct((_E,), jnp.int32),
        ],
        scratch_types=[
            pltpu.VMEM((_N,), jnp.int32),
            pltpu.VMEM((_CPAD,), jnp.int32),
            pltpu.VMEM((_CPAD,), jnp.int32),
            pltpu.VMEM((_CPAD,), jnp.int32),
            pltpu.VMEM((_CPAD,), jnp.int32),
        ],
    )
    def k(ja_hbm, ei_hbm, c_hbm, cr_hbm, ja_v, src_v, dst_v, c_v, cr_v):
        wid = lax.axis_index("s") * 2 + lax.axis_index("c")
        base = wid * _CHUNK
        src_v[pl.ds(_CPAD - 16, 16)] = jnp.zeros((16,), jnp.int32)
        dst_v[pl.ds(_CPAD - 16, 16)] = jnp.zeros((16,), jnp.int32)
        pltpu.sync_copy(ja_hbm, ja_v)
        pltpu.sync_copy(ei_hbm.at[pl.ds(base, _CHUNK)], src_v.at[pl.ds(0, _CHUNK)])
        pltpu.sync_copy(ei_hbm.at[pl.ds(_E + base, _CHUNK)], dst_v.at[pl.ds(0, _CHUNK)])

        def body(i, carry):
            sl = pl.ds(i * 16, 16)
            a_s = plsc.load_gather(ja_v, [src_v[sl]])
            a_d = plsc.load_gather(ja_v, [dst_v[sl]])
            c_v[sl] = a_s * _A + a_d
            cr_v[sl] = a_d * _A + a_s
            return carry

        lax.fori_loop(0, _CPAD // 16, body, 0, unroll=4)
        pltpu.sync_copy(c_v.at[pl.ds(0, _CHUNK)], c_hbm.at[pl.ds(base, _CHUNK)])
        pltpu.sync_copy(cr_v.at[pl.ds(0, _CHUNK)], cr_hbm.at[pl.ds(base, _CHUNK)])

    return k(joint_acts, edge_index.reshape(2 * _E))


def _edge_body(xu_ref, xr_ref, c_ref, cr_ref,
               w1_ref, b1_ref, w3_ref, b3_ref, w2_ref, b2t_ref, out_ref):
    @pl.when(pl.program_id(0) == 0)
    def _init():
        out_ref[...] = jnp.zeros_like(out_ref)

    diag = (lax.broadcasted_iota(jnp.int32, (_A * _A, _A * _A), 0)
            == lax.broadcasted_iota(jnp.int32, (_A * _A, _A * _A), 1))

    def stream(x, idx_row):
        g = jnp.maximum(
            jnp.dot(x, w1_ref[...], preferred_element_type=jnp.float32)
            + b1_ref[...], 0.0)
        g = jnp.maximum(
            jnp.dot(g, w3_ref[...], preferred_element_type=jnp.float32)
            + b3_ref[...], 0.0)
        onehot_t = jnp.where(
            idx_row == lax.broadcasted_iota(jnp.int32, (_A * _A, _BE), 0),
            1.0, 0.0)
        og = jnp.dot(onehot_t, g, preferred_element_type=jnp.float32)
        og_hi = og.astype(jnp.bfloat16).astype(jnp.float32)
        og_lo = og - og_hi
        prod = (jnp.dot(og_hi, w2_ref[...], preferred_element_type=jnp.float32)
                + jnp.dot(og_lo, w2_ref[...], preferred_element_type=jnp.float32))
        return (jnp.sum(jnp.where(diag, prod, 0.0))
                + jnp.sum(onehot_t * b2t_ref[...]))

    base = pl.multiple_of(pl.program_id(0) * _BE, 128)
    part = (stream(xu_ref[...], c_ref[pl.ds(base, _BE)].reshape(1, _BE))
            + stream(xr_ref[...], cr_ref[pl.ds(base, _BE)].reshape(1, _BE)))
    out_ref[...] = out_ref[...] + part


def _node_body(x_ref, ja_ref, edge_ref, w1_ref, b1_ref, w3_ref, b3_ref,
               w2_ref, b2_ref, out_ref):
    h = jnp.maximum(
        jnp.dot(x_ref[...], w1_ref[...], preferred_element_type=jnp.float32)
        + b1_ref[...], 0.0)
    h = jnp.maximum(
        jnp.dot(h, w3_ref[...], preferred_element_type=jnp.float32)
        + b3_ref[...], 0.0)
    logit = (jnp.dot(h, w2_ref[...], preferred_element_type=jnp.float32)
             + b2_ref[...])
    onehot_t = jnp.where(
        ja_ref[...].reshape(1, _N)
        == lax.broadcasted_iota(jnp.int32, (_A, _N), 0), 1.0, 0.0)
    hi = logit.astype(jnp.bfloat16).astype(jnp.float32)
    lo = logit - hi
    prod = (jnp.dot(onehot_t, hi, preferred_element_type=jnp.float32)
            + jnp.dot(onehot_t, lo, preferred_element_type=jnp.float32))
    diag = (lax.broadcasted_iota(jnp.int32, (_A, _A), 0)
            == lax.broadcasted_iota(jnp.int32, (_A, _A), 1))
    out_ref[...] = (jnp.sum(jnp.where(diag, prod, 0.0)).reshape(1, 1)
                    + 0.25 * edge_ref[...])


def kernel(edge_index, joint_acts, edge_feats_u, node_feats_u,
           edge_feat_reflected_u, W_ju1, b_ju1, W_ju3, b_ju3, W_ju2, b_ju2,
           W_iu1, b_iu1, W_iu3, b_iu3, W_iu2, b_iu2):
    c_flat, cr_flat = _sc_codes(joint_acts, edge_index)

    wfull = lambda shape: pl.BlockSpec(shape, lambda i: (0,) * len(shape))
    edge_out = pl.pallas_call(
        _edge_body,
        grid=(_E // _BE,),
        in_specs=[
            pl.BlockSpec((_BE, 3 * _DIM), lambda i: (i, 0)),
            pl.BlockSpec((_BE, 3 * _DIM), lambda i: (i, 0)),
            pl.BlockSpec((_E,), lambda i: (0,)),
            pl.BlockSpec((_E,), lambda i: (0,)),
            wfull((3 * _DIM, 32)),
            wfull((1, 32)),
            wfull((32, 32)),
            wfull((1, 32)),
            wfull((32, _A * _A)),
            wfull((_A * _A, 1)),
        ],
        out_specs=pl.BlockSpec((1, 1), lambda i: (0, 0)),
        out_shape=jax.ShapeDtypeStruct((1, 1), jnp.float32),
    )(edge_feats_u, edge_feat_reflected_u, c_flat, cr_flat,
      W_ju1, b_ju1.reshape(1, -1), W_ju3, b_ju3.reshape(1, -1),
      W_ju2, b_ju2.reshape(-1, 1))

    node_out = pl.pallas_call(
        _node_body,
        grid=(1,),
        in_specs=[
            pl.BlockSpec((_N, 2 * _DIM), lambda i: (0, 0)),
            pl.BlockSpec((1, 1, _N), lambda i: (0, 0, 0)),
            wfull((1, 1)),
            wfull((2 * _DIM, 32)),
            wfull((1, 32)),
            wfull((32, 32)),
            wfull((1, 32)),
            wfull((32, _A)),
            wfull((1, _A)),
        ],
        out_specs=pl.BlockSpec((1, 1), lambda i: (0, 0)),
        out_shape=jax.ShapeDtypeStruct((1, 1), jnp.float32),
    )(node_feats_u, joint_acts.reshape(1, 1, _N), edge_out,
      W_iu1, b_iu1.reshape(1, -1), W_iu3, b_iu3.reshape(1, -1),
      W_iu2, b_iu2.reshape(1, -1))

    return node_out


# speedup vs baseline: 1.4456x; 1.0044x over previous
import functools

import jax
import jax.numpy as jnp
from jax import lax
from jax.experimental import pallas as pl
from jax.experimental.pallas import tpu as pltpu
from jax.experimental.pallas import tpu_sc as plsc

_N = 10000
_E = 160000
_A = 8
_DIM = 128

_NUM_WORKERS = 32
_CHUNK = _E // _NUM_WORKERS
_CPAD = (_CHUNK + 15) // 16 * 16

_BE = 6400


def _sc_codes(joint_acts, edge_index):
    mesh = plsc.VectorSubcoreMesh(core_axis_name="c", subcore_axis_name="s")

    @functools.partial(
        pl.kernel,
        mesh=mesh,
        compiler_params=pltpu.CompilerParams(needs_layout_passes=False),
        out_type=[
            jax.ShapeDtypeStruct((_E,), jnp.int32),
            jax.ShapeDtypeStruct((_E,), jnp.int32),
        ],
        scratch_types=[
            pltpu.VMEM((_N,), jnp.int32),
            pltpu.VMEM((_CPAD,), jnp.int32),
            pltpu.VMEM((_CPAD,), jnp.int32),
            pltpu.VMEM((_CPAD,), jnp.int32),
            pltpu.VMEM((_CPAD,), jnp.int32),
        ],
    )
    def k(ja_hbm, ei_hbm, c_hbm, cr_hbm, ja_v, src_v, dst_v, c_v, cr_v):
        wid = lax.axis_index("s") * 2 + lax.axis_index("c")
        base = wid * _CHUNK
        src_v[pl.ds(_CPAD - 16, 16)] = jnp.zeros((16,), jnp.int32)
        dst_v[pl.ds(_CPAD - 16, 16)] = jnp.zeros((16,), jnp.int32)
        pltpu.sync_copy(ja_hbm, ja_v)
        pltpu.sync_copy(ei_hbm.at[pl.ds(base, _CHUNK)], src_v.at[pl.ds(0, _CHUNK)])
        pltpu.sync_copy(ei_hbm.at[pl.ds(_E + base, _CHUNK)], dst_v.at[pl.ds(0, _CHUNK)])

        def body(i, carry):
            sl = pl.ds(i * 16, 16)
            a_s = plsc.load_gather(ja_v, [src_v[sl]])
            a_d = plsc.load_gather(ja_v, [dst_v[sl]])
            c_v[sl] = a_s * _A + a_d
            cr_v[sl] = a_d * _A + a_s
            return carry

        lax.fori_loop(0, _CPAD // 16, body, 0, unroll=4)
        pltpu.sync_copy(c_v.at[pl.ds(0, _CHUNK)], c_hbm.at[pl.ds(base, _CHUNK)])
        pltpu.sync_copy(cr_v.at[pl.ds(0, _CHUNK)], cr_hbm.at[pl.ds(base, _CHUNK)])

    return k(joint_acts, edge_index.reshape(2 * _E))


def _edge_body(node_ref, xu_ref, xr_ref, c_ref, cr_ref,
               w1_ref, b1_ref, w3_ref, b3_ref, w2_ref, b2t_ref, out_ref):
    @pl.when(pl.program_id(0) == 0)
    def _init():
        out_ref[...] = node_ref[...]

    diag = (lax.broadcasted_iota(jnp.int32, (_A * _A, _A * _A), 0)
            == lax.broadcasted_iota(jnp.int32, (_A * _A, _A * _A), 1))

    def stream(x, idx_row):
        g = jnp.maximum(
            jnp.dot(x, w1_ref[...], preferred_element_type=jnp.float32)
            + b1_ref[...], 0.0)
        g = jnp.maximum(
            jnp.dot(g, w3_ref[...], preferred_element_type=jnp.float32)
            + b3_ref[...], 0.0)
        onehot_t = jnp.where(
            idx_row == lax.broadcasted_iota(jnp.int32, (_A * _A, _BE), 0),
            1.0, 0.0)
        og = jnp.dot(onehot_t, g, preferred_element_type=jnp.float32)
        og_hi = og.astype(jnp.bfloat16).astype(jnp.float32)
        og_lo = og - og_hi
        prod = (jnp.dot(og_hi, w2_ref[...], preferred_element_type=jnp.float32)
                + jnp.dot(og_lo, w2_ref[...], preferred_element_type=jnp.float32))
        return (jnp.sum(jnp.where(diag, prod, 0.0))
                + jnp.sum(onehot_t * b2t_ref[...]))

    base = pl.multiple_of(pl.program_id(0) * _BE, 128)
    part = (stream(xu_ref[...], c_ref[pl.ds(base, _BE)].reshape(1, _BE))
            + stream(xr_ref[...], cr_ref[pl.ds(base, _BE)].reshape(1, _BE)))
    out_ref[...] = out_ref[...] + 0.25 * part


def _node_body(x_ref, ja_ref, w1_ref, b1_ref, w3_ref, b3_ref,
               w2_ref, b2_ref, out_ref):
    h = jnp.maximum(
        jnp.dot(x_ref[...], w1_ref[...], preferred_element_type=jnp.float32)
        + b1_ref[...], 0.0)
    h = jnp.maximum(
        jnp.dot(h, w3_ref[...], preferred_element_type=jnp.float32)
        + b3_ref[...], 0.0)
    logit = (jnp.dot(h, w2_ref[...], preferred_element_type=jnp.float32)
             + b2_ref[...])
    onehot_t = jnp.where(
        ja_ref[...].reshape(1, _N)
        == lax.broadcasted_iota(jnp.int32, (_A, _N), 0), 1.0, 0.0)
    hi = logit.astype(jnp.bfloat16).astype(jnp.float32)
    lo = logit - hi
    prod = (jnp.dot(onehot_t, hi, preferred_element_type=jnp.float32)
            + jnp.dot(onehot_t, lo, preferred_element_type=jnp.float32))
    diag = (lax.broadcasted_iota(jnp.int32, (_A, _A), 0)
            == lax.broadcasted_iota(jnp.int32, (_A, _A), 1))
    out_ref[...] = jnp.sum(jnp.where(diag, prod, 0.0)).reshape(1, 1)


def kernel(edge_index, joint_acts, edge_feats_u, node_feats_u,
           edge_feat_reflected_u, W_ju1, b_ju1, W_ju3, b_ju3, W_ju2, b_ju2,
           W_iu1, b_iu1, W_iu3, b_iu3, W_iu2, b_iu2):
    c_flat, cr_flat = _sc_codes(joint_acts, edge_index)

    wfull = lambda shape: pl.BlockSpec(shape, lambda i: (0,) * len(shape))
    node_out = pl.pallas_call(
        _node_body,
        grid=(1,),
        in_specs=[
            pl.BlockSpec((_N, 2 * _DIM), lambda i: (0, 0)),
            pl.BlockSpec((1, 1, _N), lambda i: (0, 0, 0)),
            wfull((2 * _DIM, 32)),
            wfull((1, 32)),
            wfull((32, 32)),
            wfull((1, 32)),
            wfull((32, _A)),
            wfull((1, _A)),
        ],
        out_specs=pl.BlockSpec((1, 1), lambda i: (0, 0)),
        out_shape=jax.ShapeDtypeStruct((1, 1), jnp.float32),
    )(node_feats_u, joint_acts.reshape(1, 1, _N),
      W_iu1, b_iu1.reshape(1, -1), W_iu3, b_iu3.reshape(1, -1),
      W_iu2, b_iu2.reshape(1, -1))

    edge_out = pl.pallas_call(
        _edge_body,
        grid=(_E // _BE,),
        in_specs=[
            wfull((1, 1)),
            pl.BlockSpec((_BE, 3 * _DIM), lambda i: (i, 0)),
            pl.BlockSpec((_BE, 3 * _DIM), lambda i: (i, 0)),
            pl.BlockSpec((_E,), lambda i: (0,)),
            pl.BlockSpec((_E,), lambda i: (0,)),
            wfull((3 * _DIM, 32)),
            wfull((1, 32)),
            wfull((32, 32)),
            wfull((1, 32)),
            wfull((32, _A * _A)),
            wfull((_A * _A, 1)),
        ],
        out_specs=pl.BlockSpec((1, 1), lambda i: (0, 0)),
        out_shape=jax.ShapeDtypeStruct((1, 1), jnp.float32),
    )(node_out, edge_feats_u, edge_feat_reflected_u, c_flat, cr_flat,
      W_ju1, b_ju1.reshape(1, -1), W_ju3, b_ju3.reshape(1, -1),
      W_ju2, b_ju2.reshape(-1, 1))

    return edge_out
